# baseline (speedup 1.0000x reference)
import functools

import jax
import jax.numpy as jnp
from jax import lax
from jax.experimental import pallas as pl
from jax.experimental.pallas import tpu as pltpu
from jax.experimental.pallas import tpu_sc as plsc

D = 256
B = 160000
R = 2 * B
NC, NS = 2, 16
NW = NC * NS
ROWS_PER_W = R // NW
CHUNK = 80
NCHUNK = ROWS_PER_W // CHUNK
NBUF = 5
PRE = NBUF - 2
GROUPS = NCHUNK // NBUF

_mesh = plsc.VectorSubcoreMesh(core_axis_name="c", subcore_axis_name="s")


@functools.partial(
    pl.kernel,
    mesh=_mesh,
    out_type=jax.ShapeDtypeStruct((R, D), jnp.float32),
    scratch_types=[
        pltpu.VMEM((NCHUNK, CHUNK), jnp.int32),
        pltpu.VMEM((NBUF, CHUNK, D), jnp.float32),
    ]
    + [pltpu.SemaphoreType.DMA] * (2 * NBUF),
)
def _gather(z_hbm, idx_hbm, out_hbm, idx_v, rows, *sems):
    gsem, wsem = sems[:NBUF], sems[NBUF:]
    wid = lax.axis_index("s") * NC + lax.axis_index("c")
    pltpu.sync_copy(idx_hbm.at[wid], idx_v)
    row0 = wid * ROWS_PER_W

    def fire_gather(c, b):
        pltpu.async_copy(z_hbm.at[idx_v.at[c]], rows.at[b], gsem[b])

    def wait_gather(c, b):
        pltpu.make_async_copy(z_hbm.at[idx_v.at[c]], rows.at[b], gsem[b]).wait()

    def out_slice(c):
        return out_hbm.at[pl.ds(row0 + c * CHUNK, CHUNK), :]

    def fire_write(c, b):
        pltpu.async_copy(rows.at[b], out_slice(c), wsem[b])

    def wait_write(c, b):
        pltpu.make_async_copy(rows.at[b], out_slice(c), wsem[b]).wait()

    def step(c, b):
        wait_gather(c, b)
        fire_write(c, b)
        f = c + PRE
        if f < NCHUNK:
            bf = (b + PRE) % NBUF
            if c >= 2:
                wait_write(c - 2, bf)
            fire_gather(f, bf)

    for c in range(PRE):
        fire_gather(c, c)
    for b in range(NBUF):
        step(b, b)

    def mid_group(g, carry):
        for b in range(NBUF):
            c = g * NBUF + b
            wait_gather(c, b)
            fire_write(c, b)
            bf = (b + PRE) % NBUF
            wait_write(c - 2, bf)
            fire_gather(c + PRE, bf)
        return carry

    lax.fori_loop(1, GROUPS - 1, mid_group, 0, unroll=False)

    for b in range(NBUF):
        step((GROUPS - 1) * NBUF + b, b)
    for b in range(NBUF):
        wait_write((GROUPS - 1) * NBUF + b, b)


def kernel(z, edge_label_index):
    idx = edge_label_index.astype(jnp.int32).T.reshape(NW, NCHUNK, CHUNK)
    return _gather(z, idx).reshape(B, 2 * D)


# speedup vs baseline: 2.6444x; 2.6444x over previous
import functools

import jax
import jax.numpy as jnp
from jax import lax
from jax.experimental import pallas as pl
from jax.experimental.pallas import tpu as pltpu
from jax.experimental.pallas import tpu_sc as plsc

D = 256
B = 160000
NC, NS = 2, 16
NW = NC * NS
EDGES_PER_W = B // NW
CHUNK = 40
GROWS = 2 * CHUNK
NCHUNK = EDGES_PER_W // CHUNK
NBUF = 5
PRE = NBUF - 2
GROUPS = NCHUNK // NBUF

_mesh = plsc.VectorSubcoreMesh(core_axis_name="c", subcore_axis_name="s")


@functools.partial(
    pl.kernel,
    mesh=_mesh,
    out_type=jax.ShapeDtypeStruct((B, 2 * D), jnp.float32),
    scratch_types=[
        pltpu.VMEM((NCHUNK, GROWS), jnp.int32),
        pltpu.VMEM((NBUF, GROWS, D), jnp.float32),
    ]
    + [pltpu.SemaphoreType.DMA] * (3 * NBUF),
)
def _gather(z_hbm, idx_hbm, out_hbm, idx_v, rows, *sems):
    gsem = sems[:NBUF]
    wsem = (sems[NBUF : 2 * NBUF], sems[2 * NBUF :])
    wid = lax.axis_index("s") * NC + lax.axis_index("c")
    pltpu.sync_copy(idx_hbm.at[wid], idx_v)
    row0 = wid * EDGES_PER_W

    def fire_gather(c, b):
        pltpu.async_copy(z_hbm.at[idx_v.at[c]], rows.at[b], gsem[b])

    def wait_gather(c, b):
        pltpu.make_async_copy(z_hbm.at[idx_v.at[c]], rows.at[b], gsem[b]).wait()

    def copy_write(c, b, h):
        return pltpu.make_async_copy(
            rows.at[b, pl.ds(h * CHUNK, CHUNK)],
            out_hbm.at[pl.ds(row0 + c * CHUNK, CHUNK), pl.ds(h * D, D)],
            wsem[h][b],
        )

    def fire_write(c, b):
        copy_write(c, b, 0).start()
        copy_write(c, b, 1).start()

    def wait_write(c, b):
        copy_write(c, b, 0).wait()
        copy_write(c, b, 1).wait()

    def step(c, b):
        wait_gather(c, b)
        fire_write(c, b)
        f = c + PRE
        if f < NCHUNK:
            bf = (b + PRE) % NBUF
            if c >= 2:
                wait_write(c - 2, bf)
            fire_gather(f, bf)

    for c in range(PRE):
        fire_gather(c, c)
    for b in range(NBUF):
        step(b, b)

    def mid_group(g, carry):
        for b in range(NBUF):
            c = g * NBUF + b
            wait_gather(c, b)
            fire_write(c, b)
            bf = (b + PRE) % NBUF
            wait_write(c - 2, bf)
            fire_gather(c + PRE, bf)
        return carry

    lax.fori_loop(1, GROUPS - 1, mid_group, 0, unroll=False)

    for b in range(NBUF):
        step((GROUPS - 1) * NBUF + b, b)
    for b in range(NBUF):
        wait_write((GROUPS - 1) * NBUF + b, b)


def kernel(z, edge_label_index):
    idx = edge_label_index.astype(jnp.int32).reshape(2, NW, NCHUNK, CHUNK)
    idx = idx.transpose(1, 2, 0, 3).reshape(NW, NCHUNK, GROWS)
    return _gather(z, idx)


# speedup vs baseline: 2.6489x; 1.0017x over previous
import functools

import jax
import jax.numpy as jnp
from jax import lax
from jax.experimental import pallas as pl
from jax.experimental.pallas import tpu as pltpu
from jax.experimental.pallas import tpu_sc as plsc

D = 256
B = 160000
NC, NS = 2, 16
NW = NC * NS
EDGES_PER_W = B // NW
CHUNK = 40
GROWS = 2 * CHUNK
NCHUNK = EDGES_PER_W // CHUNK
NBUF = 5
PRE = 2
LAG = NBUF - PRE
GROUPS = NCHUNK // NBUF

_mesh = plsc.VectorSubcoreMesh(core_axis_name="c", subcore_axis_name="s")


@functools.partial(
    pl.kernel,
    mesh=_mesh,
    out_type=jax.ShapeDtypeStruct((B, 2 * D), jnp.float32),
    scratch_types=[
        pltpu.VMEM((NCHUNK, GROWS), jnp.int32),
        pltpu.VMEM((NBUF, GROWS, D), jnp.float32),
    ]
    + [pltpu.SemaphoreType.DMA] * (3 * NBUF),
)
def _gather(z_hbm, idx_hbm, out_hbm, idx_v, rows, *sems):
    gsem = sems[:NBUF]
    wsem = (sems[NBUF : 2 * NBUF], sems[2 * NBUF :])
    wid = lax.axis_index("s") * NC + lax.axis_index("c")
    pltpu.sync_copy(idx_hbm.at[wid], idx_v)
    row0 = wid * EDGES_PER_W

    def fire_gather(c, b):
        pltpu.async_copy(z_hbm.at[idx_v.at[c]], rows.at[b], gsem[b])

    def wait_gather(c, b):
        pltpu.make_async_copy(z_hbm.at[idx_v.at[c]], rows.at[b], gsem[b]).wait()

    def copy_write(c, b, h):
        return pltpu.make_async_copy(
            rows.at[b, pl.ds(h * CHUNK, CHUNK)],
            out_hbm.at[pl.ds(row0 + c * CHUNK, CHUNK), pl.ds(h * D, D)],
            wsem[h][b],
        )

    def fire_write(c, b):
        copy_write(c, b, 0).start()
        copy_write(c, b, 1).start()

    def wait_write(c, b):
        copy_write(c, b, 0).wait()
        copy_write(c, b, 1).wait()

    def step(c, b):
        wait_gather(c, b)
        fire_write(c, b)
        f = c + PRE
        if f < NCHUNK:
            bf = (b + PRE) % NBUF
            if c >= LAG:
                wait_write(c - LAG, bf)
            fire_gather(f, bf)

    for c in range(PRE):
        fire_gather(c, c)
    for b in range(NBUF):
        step(b, b)

    def mid_group(g, carry):
        for b in range(NBUF):
            c = g * NBUF + b
            wait_gather(c, b)
            fire_write(c, b)
            bf = (b + PRE) % NBUF
            wait_write(c - LAG, bf)
            fire_gather(c + PRE, bf)
        return carry

    lax.fori_loop(1, GROUPS - 1, mid_group, 0, unroll=False)

    for b in range(NBUF):
        step((GROUPS - 1) * NBUF + b, b)
    for b in range(NBUF):
        wait_write((GROUPS - 1) * NBUF + b, b)


def kernel(z, edge_label_index):
    idx = edge_label_index.astype(jnp.int32).reshape(2, NW, NCHUNK, CHUNK)
    idx = idx.transpose(1, 2, 0, 3).reshape(NW, NCHUNK, GROWS)
    return _gather(z, idx)
